# baseline (speedup 1.0000x reference)
import functools

import jax
import jax.numpy as jnp
from jax import lax
from jax.experimental import pallas as pl
from jax.experimental.pallas import tpu as pltpu
from jax.experimental.pallas import tpu_sc as plsc

_N, _C, _D, _H, _W = 2, 4, 64, 128, 128
_V = _D * _H * _W
_LANES = 128
_ROWS = _V // _LANES
_EPS = 0.0001

_TC_ROWS = 7168
_BR = 4096

_NC, _NS = 2, 16
_NW = _NC * _NS
_CH = 4096


def _accumulate(x, t, acc_ref, which, n):
    cidx = jax.lax.broadcasted_iota(jnp.int32, (_C, 1, 1), 0)
    mask = t[None] == cidx
    totals = jnp.sum(x, axis=1)
    sel = jnp.sum(jnp.where(mask, x, 0.0), axis=1)
    cnt = jnp.sum(mask.astype(jnp.float32), axis=1)
    acc_ref[0, which, n] += totals
    acc_ref[1, which, n] += sel
    acc_ref[2, which, n] += cnt


def _sums_body(x1_ref, t1_ref, x2_ref, t2_ref, out_ref, acc_ref):
    ti = pl.program_id(0)
    n = pl.program_id(1)
    g = pl.program_id(2)
    ng = pl.num_programs(2)

    @pl.when((ti == 0) & (n == 0) & (g == 0))
    def _():
        acc_ref[...] = jnp.zeros_like(acc_ref)

    @pl.when(ti == 0)
    def _():
        _accumulate(x1_ref[0], t1_ref[0], acc_ref, 0, n)

    @pl.when(ti == 1)
    def _():
        _accumulate(x2_ref[0], t2_ref[0], acc_ref, 1, n)

    @pl.when((ti == 1) & (n == _N - 1) & (g == ng - 1))
    def _():
        out_ref[...] = jnp.sum(acc_ref[...], axis=-1)


_BRS = 1024
_K = 4


def _tc_dice_sums_stream(x1, t1, x2, t2, rows):
    ngc = rows // _BRS
    assert rows % _BRS == 0
    ns = 2 * _N * ngc

    def _body(x1_ref, t1_ref, x2_ref, t2_ref, out_ref, rx, rt, acc, sems):
        s = pl.program_id(0)

        def decomp(step):
            p = step // (_N * ngc)
            rem = lax.rem(step, _N * ngc)
            return p, rem // ngc, lax.rem(rem, ngc)

        def issue(step):
            p, n, g = decomp(step)
            slot = lax.rem(step, _K)

            @pl.when(p == 0)
            def _():
                pltpu.make_async_copy(
                    x1_ref.at[n, :, pl.ds(g * _BRS, _BRS), :],
                    rx.at[slot], sems.at[slot, 0]).start()
                pltpu.make_async_copy(
                    t1_ref.at[n, pl.ds(g * _BRS, _BRS), :],
                    rt.at[slot], sems.at[slot, 1]).start()

            @pl.when(p == 1)
            def _():
                pltpu.make_async_copy(
                    x2_ref.at[n, :, pl.ds(g * _BRS, _BRS), :],
                    rx.at[slot], sems.at[slot, 0]).start()
                pltpu.make_async_copy(
                    t2_ref.at[n, pl.ds(g * _BRS, _BRS), :],
                    rt.at[slot], sems.at[slot, 1]).start()

        @pl.when(s == 0)
        def _():
            acc[...] = jnp.zeros_like(acc)
            for k in range(_K - 1):
                issue(jnp.int32(k))

        @pl.when(s + _K - 1 < ns)
        def _():
            issue(s + _K - 1)

        slot = lax.rem(s, _K)
        pltpu.make_async_copy(
            x1_ref.at[0, :, pl.ds(0, _BRS), :], rx.at[slot],
            sems.at[slot, 0]).wait()
        pltpu.make_async_copy(
            t1_ref.at[0, pl.ds(0, _BRS), :], rt.at[slot],
            sems.at[slot, 1]).wait()

        p, n, g = decomp(s)
        x = rx[slot]
        t = rt[slot]
        cidx = jax.lax.broadcasted_iota(jnp.int32, (_C, 1, 1), 0)
        mask = t[None] == cidx
        acc[0, p, n] += jnp.sum(x, axis=1)
        acc[1, p, n] += jnp.sum(jnp.where(mask, x, 0.0), axis=1)
        acc[2, p, n] += jnp.sum(mask.astype(jnp.float32), axis=1)

        @pl.when(s == ns - 1)
        def _():
            out_ref[...] = jnp.sum(acc[...], axis=-1)

    return pl.pallas_call(
        _body,
        grid=(ns,),
        in_specs=[pl.BlockSpec(memory_space=pl.ANY)] * 4,
        out_specs=pl.BlockSpec((3, 2, _N, _C), lambda s: (0, 0, 0, 0)),
        out_shape=jax.ShapeDtypeStruct((3, 2, _N, _C), jnp.float32),
        scratch_shapes=[
            pltpu.VMEM((_K, _C, _BRS, _LANES), jnp.float32),
            pltpu.VMEM((_K, _BRS, _LANES), jnp.int32),
            pltpu.VMEM((3, 2, _N, _C, _LANES), jnp.float32),
            pltpu.SemaphoreType.DMA((_K, 2)),
        ],
    )(x1, t1, x2, t2)


def _tc_dice_sums(x1, t1, x2, t2, rows):
    br = next(b for b in range(min(_BR, rows), 7, -8) if rows % b == 0)
    ng = rows // br
    last = (_N - 1, 0, ng - 1, 0)

    def x_spec(which):
        def idx(ti, n, g):
            on = ti == which
            pin = last if which == 0 else (0, 0, 0, 0)
            return (jnp.where(on, n, pin[0]), 0, jnp.where(on, g, pin[2]), 0)
        return pl.BlockSpec((1, _C, br, _LANES), idx)

    def t_spec(which):
        def idx(ti, n, g):
            on = ti == which
            pin = last if which == 0 else (0, 0, 0, 0)
            return (jnp.where(on, n, pin[0]), jnp.where(on, g, pin[2]), 0)
        return pl.BlockSpec((1, br, _LANES), idx)

    sums = pl.pallas_call(
        _sums_body,
        grid=(2, _N, ng),
        in_specs=[x_spec(0), t_spec(0), x_spec(1), t_spec(1)],
        out_specs=pl.BlockSpec(
            (3, 2, _N, _C), lambda ti, n, g: (0, 0, 0, 0)),
        out_shape=jax.ShapeDtypeStruct((3, 2, _N, _C), jnp.float32),
        scratch_shapes=[pltpu.VMEM((3, 2, _N, _C, _LANES), jnp.float32)],
    )(x1, t1, x2, t2)
    return sums


def _sc_dice_sums(x1, t1, x2, t2, row_start):
    rows_sc = _ROWS - row_start
    rpw = rows_sc // _NW
    assert rows_sc % _NW == 0
    rr = next(r for r in range(min(128, rpw), 0, -1) if rpw % r == 0)
    nch = rpw // rr

    mesh = plsc.VectorSubcoreMesh(core_axis_name="c", subcore_axis_name="s")

    @functools.partial(
        pl.kernel,
        out_type=jax.ShapeDtypeStruct((_NW, 768), jnp.float32),
        mesh=mesh,
        scratch_types=[
            pltpu.VMEM((rr, _LANES), jnp.int32),
            pltpu.VMEM((rr, _LANES), jnp.float32),
            pltpu.VMEM((rr, _LANES), jnp.float32),
            pltpu.VMEM((rr, _LANES), jnp.float32),
            pltpu.VMEM((rr, _LANES), jnp.float32),
            pltpu.VMEM((768,), jnp.float32),
            pltpu.SemaphoreType.DMA,
        ],
    )
    def sck(x1_hbm, t1_hbm, x2_hbm, t2_hbm, out_hbm,
            gt_v, xb0, xb1, xb2, xb3, res_v, sem):
        wid = lax.axis_index("s") * _NC + lax.axis_index("c")
        xbufs = (xb0, xb1, xb2, xb3)
        for t_i, (xh, th) in enumerate(((x1_hbm, t1_hbm), (x2_hbm, t2_hbm))):
            for n in range(_N):
                accs = tuple(jnp.zeros((16,), jnp.float32) for _ in range(12))
                for k in range(nch):
                    r0 = row_start + wid * rpw + k * rr
                    cps = [
                        pltpu.async_copy(
                            xh.at[n, c, pl.ds(r0, rr), :], xbufs[c], sem)
                        for c in range(_C)
                    ]
                    cpg = pltpu.async_copy(
                        th.at[n, pl.ds(r0, rr), :], gt_v, sem)
                    for cp in cps:
                        cp.wait()
                    cpg.wait()

                    def body(r, a):
                        new = list(a)
                        for w0 in range(0, _LANES, 16):
                            g = gt_v[r, pl.ds(w0, 16)]
                            for c in range(_C):
                                x = xbufs[c][r, pl.ds(w0, 16)]
                                m = g == c
                                new[c] = new[c] + x
                                new[4 + c] = new[4 + c] + jnp.where(m, x, 0.0)
                                new[8 + c] = (
                                    new[8 + c] + jnp.where(m, 1.0, 0.0))
                        return tuple(new)

                    accs = plsc.parallel_loop(0, rr, 1, carry=accs)(body)
                base = (t_i * _N + n) * 12
                for j in range(12):
                    res_v[pl.ds((base + j) * 16, 16)] = accs[j]
        pltpu.sync_copy(res_v, out_hbm.at[wid])

    rows = sck(x1, t1, x2, t2)
    packed = rows.reshape(_NW, 48, 16).sum((0, 2)).reshape(2, _N, 3, _C)
    return packed.transpose(0, 2, 1, 3)


def _dice_loss(sums, weights):
    total, sel, cnt = sums[0], sums[1], sums[2]
    numerator = 2.0 * sel
    denominator = total + cnt + _EPS
    loss_per_channel = weights * (1.0 - numerator / denominator)
    return loss_per_channel.sum() / _N


def kernel(output, gt, shape_output, shape_gt, class_weights):
    out_f = output.reshape(_N, _C, _ROWS, _LANES)
    sout_f = shape_output.reshape(_N, _C, _ROWS, _LANES)
    gt_f = gt.reshape(_N, _ROWS, _LANES).astype(jnp.int32)
    sgt_f = shape_gt.reshape(_N, _ROWS, _LANES).astype(jnp.int32)

    sums_a = jnp.zeros((3, _N, _C), jnp.float32)
    sums_b = jnp.zeros((3, _N, _C), jnp.float32)
    if _TC_ROWS < _ROWS:
        sc = _sc_dice_sums(out_f, gt_f, sout_f, sgt_f, _TC_ROWS)
        sums_a = sums_a + sc[0]
        sums_b = sums_b + sc[1]
    if _TC_ROWS > 0:
        tc = _tc_dice_sums_stream(out_f, gt_f, sout_f, sgt_f, _TC_ROWS)
        sums_a = sums_a + tc[:, 0]
        sums_b = sums_b + tc[:, 1]

    loss_a = _dice_loss(sums_a, class_weights)
    loss_b = _dice_loss(sums_b, class_weights)
    return (loss_a, loss_b)


# speedup vs baseline: 1.5005x; 1.5005x over previous
import jax
import jax.numpy as jnp
from jax import lax
from jax.experimental import pallas as pl
from jax.experimental.pallas import tpu as pltpu

_N, _C, _D, _H, _W = 2, 4, 64, 128, 128
_V = _D * _H * _W
_LANES = 128
_ROWS = _V // _LANES
_EPS = 0.0001

_BRS = 1024
_K = 4


def _dice_sums_stream(x1, t1, x2, t2):
    ngc = _ROWS // _BRS
    ns = 2 * _N * ngc

    def _body(x1_ref, t1_ref, x2_ref, t2_ref, out_ref, rx, rt, acc, sems):
        s = pl.program_id(0)

        def decomp(step):
            p = step // (_N * ngc)
            rem = lax.rem(step, _N * ngc)
            return p, rem // ngc, lax.rem(rem, ngc)

        def issue(step):
            p, n, g = decomp(step)
            slot = lax.rem(step, _K)

            @pl.when(p == 0)
            def _():
                pltpu.make_async_copy(
                    x1_ref.at[n, :, pl.ds(g * _BRS, _BRS), :],
                    rx.at[slot], sems.at[slot, 0]).start()
                pltpu.make_async_copy(
                    t1_ref.at[n, pl.ds(g * _BRS, _BRS), :],
                    rt.at[slot], sems.at[slot, 1]).start()

            @pl.when(p == 1)
            def _():
                pltpu.make_async_copy(
                    x2_ref.at[n, :, pl.ds(g * _BRS, _BRS), :],
                    rx.at[slot], sems.at[slot, 0]).start()
                pltpu.make_async_copy(
                    t2_ref.at[n, pl.ds(g * _BRS, _BRS), :],
                    rt.at[slot], sems.at[slot, 1]).start()

        @pl.when(s == 0)
        def _():
            acc[...] = jnp.zeros_like(acc)
            for k in range(_K - 1):
                issue(jnp.int32(k))

        @pl.when(s + _K - 1 < ns)
        def _():
            issue(s + _K - 1)

        slot = lax.rem(s, _K)
        pltpu.make_async_copy(
            x1_ref.at[0, :, pl.ds(0, _BRS), :], rx.at[slot],
            sems.at[slot, 0]).wait()
        pltpu.make_async_copy(
            t1_ref.at[0, pl.ds(0, _BRS), :], rt.at[slot],
            sems.at[slot, 1]).wait()

        p, n, g = decomp(s)
        x = rx[slot]
        t = rt[slot]
        cidx = jax.lax.broadcasted_iota(jnp.int32, (_C, 1, 1), 0)
        mask = t[None] == cidx
        acc[0, p, n] += jnp.sum(x, axis=1)
        acc[1, p, n] += jnp.sum(jnp.where(mask, x, 0.0), axis=1)
        acc[2, p, n] += jnp.sum(mask.astype(jnp.float32), axis=1)

        @pl.when(s == ns - 1)
        def _():
            out_ref[...] = jnp.sum(acc[...], axis=-1)

    return pl.pallas_call(
        _body,
        grid=(ns,),
        in_specs=[pl.BlockSpec(memory_space=pl.ANY)] * 4,
        out_specs=pl.BlockSpec((3, 2, _N, _C), lambda s: (0, 0, 0, 0)),
        out_shape=jax.ShapeDtypeStruct((3, 2, _N, _C), jnp.float32),
        scratch_shapes=[
            pltpu.VMEM((_K, _C, _BRS, _LANES), jnp.float32),
            pltpu.VMEM((_K, _BRS, _LANES), jnp.int32),
            pltpu.VMEM((3, 2, _N, _C, _LANES), jnp.float32),
            pltpu.SemaphoreType.DMA((_K, 2)),
        ],
    )(x1, t1, x2, t2)


def _dice_loss(sums, weights):
    total, sel, cnt = sums[0], sums[1], sums[2]
    numerator = 2.0 * sel
    denominator = total + cnt + _EPS
    loss_per_channel = weights * (1.0 - numerator / denominator)
    return loss_per_channel.sum() / _N


def kernel(output, gt, shape_output, shape_gt, class_weights):
    out_f = output.reshape(_N, _C, _ROWS, _LANES)
    sout_f = shape_output.reshape(_N, _C, _ROWS, _LANES)
    gt_f = gt.reshape(_N, _ROWS, _LANES).astype(jnp.int32)
    sgt_f = shape_gt.reshape(_N, _ROWS, _LANES).astype(jnp.int32)

    sums = _dice_sums_stream(out_f, gt_f, sout_f, sgt_f)
    loss_a = _dice_loss(sums[:, 0], class_weights)
    loss_b = _dice_loss(sums[:, 1], class_weights)
    return (loss_a, loss_b)
